# baseline (speedup 1.0000x reference)
import functools

import jax
import jax.numpy as jnp
from jax.experimental import pallas as pl
from jax.experimental.pallas import tpu as pltpu

K = 20
LANES = 128


def _edge_kernel(x_blk_ref, x_all_ref, out_ref, *, n, d, k):
    x = x_blk_ref[0]
    xa = x_all_ref[0]
    block = x.shape[0]
    nc = n // LANES

    inner = jnp.dot(x, xa.T, preferred_element_type=jnp.float32)
    xsq = jnp.sum(x * x, axis=1, keepdims=True)
    xasq = jnp.sum(xa * xa, axis=1, keepdims=True).T
    dist = (xsq + (-2.0 * inner) + xasq).reshape(block, nc, LANES)

    shape3 = (block, nc, LANES)
    iota3 = (jax.lax.broadcasted_iota(jnp.int32, shape3, 1) * LANES
             + jax.lax.broadcasted_iota(jnp.int32, shape3, 2)
             ).astype(jnp.float32)
    nf = jnp.float32(n)
    inf = jnp.float32(jnp.inf)

    def argmin_lex(dm):
        runval, runidx = dm[:, 0, :], iota3[:, 0, :]
        for c in range(1, nc):
            cond = dm[:, c, :] < runval
            runidx = jnp.where(cond, iota3[:, c, :], runidx)
            runval = jnp.minimum(dm[:, c, :], runval)
        mval = jnp.min(runval, axis=-1, keepdims=True)
        return jnp.min(jnp.where(runval == mval, runidx, nf),
                       axis=-1, keepdims=True)

    first = argmin_lex(dist)
    dm = dist
    neighbors = []
    for r in range(k):
        sel = iota3 == first[:, :, None]
        oh = sel.astype(jnp.float32).reshape(block, n)
        neighbors.append(jnp.dot(oh, xa, preferred_element_type=jnp.float32))
        if r < k - 1:
            dm = jnp.where(sel, inf, dm)
            first = argmin_lex(dm)

    for j in range(k):
        base = j * 2 * d
        out_ref[0, :, base:base + d] = x
        out_ref[0, :, base + d:base + 2 * d] = neighbors[j] - x


def kernel(inputs):
    b, n, d = inputs.shape
    block = 512
    grid = (b, n // block)

    out = pl.pallas_call(
        functools.partial(_edge_kernel, n=n, d=d, k=K),
        grid=grid,
        in_specs=[
            pl.BlockSpec((1, block, d), lambda i, j: (i, j, 0)),
            pl.BlockSpec((1, n, d), lambda i, j: (i, 0, 0)),
        ],
        out_specs=pl.BlockSpec((1, block, 2 * d * K), lambda i, j: (i, j, 0)),
        out_shape=jax.ShapeDtypeStruct((b, n, 2 * d * K), jnp.float32),
    )(inputs, inputs)
    return out.reshape(b, n, K, 2 * d)


# speedup vs baseline: 6.9137x; 6.9137x over previous
import functools

import jax
import jax.numpy as jnp
from jax.experimental import pallas as pl
from jax.experimental.pallas import tpu as pltpu

K = 20
LANES = 128


def _edge_kernel(x_blk_ref, x_all_ref, out_ref, *, n, d, k):
    x = x_blk_ref[0]
    xa = x_all_ref[0]
    block = x.shape[0]
    nc = n // LANES

    inner = jnp.dot(x, xa.T, preferred_element_type=jnp.float32)
    xsq = jnp.sum(x * x, axis=1, keepdims=True)
    xasq = jnp.sum(xa * xa, axis=1, keepdims=True).T
    dist = xsq + (-2.0 * inner) + xasq

    iota = jax.lax.broadcasted_iota(
        jnp.int32, (block, n), 1).astype(jnp.float32)
    nf = jnp.float32(n)
    inf = jnp.float32(jnp.inf)

    def argmin_lex(dm):
        runval = dm[:, 0:LANES]
        runidx = iota[:, 0:LANES]
        for c in range(1, nc):
            dc = dm[:, c * LANES:(c + 1) * LANES]
            cond = dc < runval
            runidx = jnp.where(cond, iota[:, c * LANES:(c + 1) * LANES], runidx)
            runval = jnp.minimum(dc, runval)
        mval = jnp.min(runval, axis=-1, keepdims=True)
        return jnp.min(jnp.where(runval == mval, runidx, nf),
                       axis=-1, keepdims=True)

    first = argmin_lex(dist)
    dm = dist
    neighbors = []
    for r in range(k):
        sel = iota == first
        oh = sel.astype(jnp.float32)
        neighbors.append(jnp.dot(oh, xa, preferred_element_type=jnp.float32))
        if r < k - 1:
            dm = jnp.where(sel, inf, dm)
            first = argmin_lex(dm)

    for j in range(k):
        base = j * 2 * d
        out_ref[0, :, base:base + d] = x
        out_ref[0, :, base + d:base + 2 * d] = neighbors[j] - x


def kernel(inputs):
    b, n, d = inputs.shape
    block = 512
    grid = (b, n // block)

    out = pl.pallas_call(
        functools.partial(_edge_kernel, n=n, d=d, k=K),
        grid=grid,
        in_specs=[
            pl.BlockSpec((1, block, d), lambda i, j: (i, j, 0)),
            pl.BlockSpec((1, n, d), lambda i, j: (i, 0, 0)),
        ],
        out_specs=pl.BlockSpec((1, block, 2 * d * K), lambda i, j: (i, j, 0)),
        out_shape=jax.ShapeDtypeStruct((b, n, 2 * d * K), jnp.float32),
    )(inputs, inputs)
    return out.reshape(b, n, K, 2 * d)
